# Initial kernel scaffold; baseline (speedup 1.0000x reference)
import jax
import jax.numpy as jnp
from jax.experimental import pallas as pl


def kernel(x, edge_index, W1, b1, W2, b2, Wf1, bf1, Wf2, bf2, Wf3, bf3, Ws, bs):
    raise NotImplementedError("write your pallas kernel here")


# speedup vs baseline: 7.1776x; 7.1776x over previous
import functools

import jax
import jax.numpy as jnp
from jax import lax
from jax.experimental import pallas as pl
from jax.experimental.pallas import tpu as pltpu
from jax.experimental.pallas import tpu_sc as plsc

N = 10000
E = 320000
D = 128

NW = 32
CH = 128
NCH = 80
EPT = NCH * CH
EPAD = NW * EPT
NPAD = 10112
RPT = NPAD // 16

_mesh = plsc.VectorSubcoreMesh(core_axis_name="c", subcore_axis_name="s")


def _zero_vmem(ref, rows, lanes):
    def zr(i, _):
        def zc(j, _):
            ref[i, pl.ds(j * 16, 16)] = jnp.zeros((16,), jnp.float32)
            return 0
        return lax.fori_loop(0, lanes // 16, zc, 0)
    lax.fori_loop(0, rows, zr, 0)


@functools.partial(
    pl.kernel,
    mesh=_mesh,
    out_type=jax.ShapeDtypeStruct((2, NPAD, D), jnp.float32),
    scratch_types=[
        pltpu.VMEM((NCH, CH), jnp.int32),
        pltpu.VMEM((CH, D), jnp.float32),
        pltpu.VMEM_SHARED((NPAD, D), jnp.float32),
    ],
)
def _deg_sc(dst_hbm, out_hbm, idx_v, ones_v, acc):
    cid = lax.axis_index("c")
    sid = lax.axis_index("s")
    wid = sid * 2 + cid
    _zero_vmem(ones_v, CH, D)

    base = sid * RPT
    for k in range(RPT // 128):
        pltpu.sync_copy(ones_v, acc.at[pl.ds(base + k * 128, 128)])
    rem = RPT % 128
    if rem:
        pltpu.sync_copy(ones_v.at[pl.ds(0, rem)],
                        acc.at[pl.ds(base + (RPT // 128) * 128, rem)])

    def onerow(i, _):
        def onecol(j, _):
            ones_v[i, pl.ds(j * 16, 16)] = jnp.ones((16,), jnp.float32)
            return 0
        return lax.fori_loop(0, D // 16, onecol, 0)
    lax.fori_loop(0, CH, onerow, 0)
    pltpu.sync_copy(dst_hbm.at[wid], idx_v)
    plsc.subcore_barrier()

    def chunk(j, _):
        pltpu.sync_copy(ones_v, acc.at[idx_v.at[j]], add=True)
        return 0
    lax.fori_loop(0, NCH, chunk, 0)

    plsc.subcore_barrier()
    pltpu.sync_copy(acc.at[pl.ds(base, RPT)], out_hbm.at[cid, pl.ds(base, RPT)])


@functools.partial(
    pl.kernel,
    mesh=_mesh,
    out_type=jax.ShapeDtypeStruct((2, NPAD, D), jnp.float32),
    scratch_types=[
        pltpu.VMEM((NCH, CH), jnp.int32),
        pltpu.VMEM((NCH, CH), jnp.int32),
        pltpu.VMEM((CH, D), jnp.float32),
        pltpu.VMEM_SHARED((NPAD, D), jnp.float32),
        pltpu.SemaphoreType.DMA,
    ],
)
def _scat_sc(g_hbm, src_hbm, dst_hbm, out_hbm,
             isrc, idst, buf0, acc, sem0):
    cid = lax.axis_index("c")
    sid = lax.axis_index("s")
    wid = sid * 2 + cid
    _zero_vmem(buf0, CH, D)
    base = sid * RPT
    for k in range(RPT // 128):
        pltpu.sync_copy(buf0, acc.at[pl.ds(base + k * 128, 128)])
    rem = RPT % 128
    if rem:
        pltpu.sync_copy(buf0.at[pl.ds(0, rem)],
                        acc.at[pl.ds(base + (RPT // 128) * 128, rem)])
    pltpu.sync_copy(src_hbm.at[wid], isrc)
    pltpu.sync_copy(dst_hbm.at[wid], idst)
    plsc.subcore_barrier()

    def chunk(j, _):
        pltpu.async_copy(g_hbm.at[isrc.at[j]], buf0, sem0).wait()
        pltpu.sync_copy(buf0, acc.at[idst.at[j]], add=True)
        return 0
    lax.fori_loop(0, NCH, chunk, 0)

    plsc.subcore_barrier()
    pltpu.sync_copy(acc.at[pl.ds(base, RPT)], out_hbm.at[cid, pl.ds(base, RPT)])


BM = 400
GRID = N // BM


def _tc1_body(x_ref, c0_ref, c1_ref, w_ref, b_ref, o_ref, inv_ref):
    cnt = (jnp.max(c0_ref[...], axis=1, keepdims=True)
           + jnp.max(c1_ref[...], axis=1, keepdims=True))
    inv = 1.0 / jnp.sqrt(cnt + 1.0)
    inv_ref[...] = jnp.broadcast_to(inv, inv_ref.shape)
    hw = jnp.dot(x_ref[...], w_ref[...], preferred_element_type=jnp.float32)
    o_ref[...] = (hw + b_ref[...]) * inv


def _tc2_body(p0_ref, p1_ref, g_ref, inv_ref, w_ref, b_ref, o_ref):
    inv = inv_ref[...]
    h1 = jax.nn.relu(inv * (p0_ref[...] + p1_ref[...] + g_ref[...]))
    hw = jnp.dot(h1, w_ref[...], preferred_element_type=jnp.float32)
    o_ref[...] = (hw + b_ref[...]) * inv


def _tc3_body(p0_ref, p1_ref, g_ref, inv_ref,
              wf1_ref, bf1_ref, wf2_ref, bf2_ref, wf3_ref, bf3_ref,
              ws_ref, bs_ref, o_ref):
    hx = inv_ref[...] * (p0_ref[...] + p1_ref[...] + g_ref[...])
    h = jax.nn.relu(jnp.dot(hx, wf1_ref[...], preferred_element_type=jnp.float32) + bf1_ref[...])
    h = jax.nn.relu(jnp.dot(h, wf2_ref[...], preferred_element_type=jnp.float32) + bf2_ref[...])
    h = jax.nn.relu(jnp.dot(h, wf3_ref[...], preferred_element_type=jnp.float32) + bf3_ref[...])
    o_ref[...] = h + jnp.dot(hx, ws_ref[...], preferred_element_type=jnp.float32) + bs_ref[...]


def _row_spec(shape):
    return pl.BlockSpec((BM, shape), lambda i: (i, 0))


def _full_spec(r, c):
    return pl.BlockSpec((r, c), lambda i: (0, 0))


_out_nd = jax.ShapeDtypeStruct((N, D), jnp.float32)
_out_row = pl.BlockSpec((BM, D), lambda i: (i, 0))

_tc1 = pl.pallas_call(
    _tc1_body, grid=(GRID,),
    in_specs=[_row_spec(D), _row_spec(D), _row_spec(D),
              _full_spec(D, D), _full_spec(1, D)],
    out_specs=[_out_row, _out_row], out_shape=[_out_nd, _out_nd])

_tc2 = pl.pallas_call(
    _tc2_body, grid=(GRID,),
    in_specs=[_row_spec(D), _row_spec(D), _row_spec(D), _row_spec(D),
              _full_spec(D, D), _full_spec(1, D)],
    out_specs=_out_row, out_shape=_out_nd)

_tc3 = pl.pallas_call(
    _tc3_body, grid=(GRID,),
    in_specs=[_row_spec(D), _row_spec(D), _row_spec(D), _row_spec(D)]
             + [_full_spec(D, D), _full_spec(1, D)] * 4,
    out_specs=_out_row, out_shape=_out_nd)


def kernel(x, edge_index, W1, b1, W2, b2, Wf1, bf1, Wf2, bf2, Wf3, bf3, Ws, bs):
    pad = EPAD - E
    src = jnp.concatenate([edge_index[0], jnp.zeros((pad,), jnp.int32)])
    dst = jnp.concatenate([edge_index[1], jnp.full((pad,), N, jnp.int32)])
    src_r = src.reshape(NW, NCH, CH)
    dst_r = dst.reshape(NW, NCH, CH)

    cnt = _deg_sc(dst_r)
    c0, c1 = cnt[0], cnt[1]

    b1r = b1.reshape(1, D)
    g1, invb = _tc1(x, c0, c1, W1, b1r)
    P1 = _scat_sc(g1, src_r, dst_r)
    g2 = _tc2(P1[0], P1[1], g1, invb, W2, b2.reshape(1, D))
    P2 = _scat_sc(g2, src_r, dst_r)
    out = _tc3(P2[0], P2[1], g2, invb,
               Wf1, bf1.reshape(1, D), Wf2, bf2.reshape(1, D),
               Wf3, bf3.reshape(1, D), Ws, bs.reshape(1, D))
    return out


# speedup vs baseline: 7.4615x; 1.0396x over previous
import functools

import jax
import jax.numpy as jnp
from jax import lax
from jax.experimental import pallas as pl
from jax.experimental.pallas import tpu as pltpu
from jax.experimental.pallas import tpu_sc as plsc

N = 10000
E = 320000
D = 128

NW = 32
CH = 128
NCH = 80
EPT = NCH * CH
EPAD = NW * EPT
NPAD = 10112
RPT = NPAD // 16

_mesh = plsc.VectorSubcoreMesh(core_axis_name="c", subcore_axis_name="s")


def _zero_vmem(ref, rows, lanes):
    def zr(i, _):
        def zc(j, _):
            ref[i, pl.ds(j * 16, 16)] = jnp.zeros((16,), jnp.float32)
            return 0
        return lax.fori_loop(0, lanes // 16, zc, 0)
    lax.fori_loop(0, rows, zr, 0)


def _decode(packed_v, j, dst_only, isrc_v, idst_v):
    def it(i, _):
        v = packed_v[j, pl.ds(i * 16, 16)]
        if not dst_only:
            isrc_v[pl.ds(i * 16, 16)] = v & 0xFFFF
        idst_v[pl.ds(i * 16, 16)] = v >> 16
        return 0
    lax.fori_loop(0, CH // 16, it, 0)


@functools.partial(
    pl.kernel,
    mesh=_mesh,
    out_type=jax.ShapeDtypeStruct((2, NPAD, D), jnp.float32),
    scratch_types=[
        pltpu.VMEM((NCH, CH), jnp.int32),
        pltpu.VMEM((CH,), jnp.int32),
        pltpu.VMEM((CH, D), jnp.float32),
        pltpu.VMEM_SHARED((NPAD, D), jnp.float32),
    ],
)
def _deg_sc(packed_hbm, out_hbm, idx_v, idst_v, ones_v, acc):
    cid = lax.axis_index("c")
    sid = lax.axis_index("s")
    wid = sid * 2 + cid
    _zero_vmem(ones_v, CH, D)

    base = sid * RPT
    zrows = 0
    while zrows < RPT:
        zc = min(CH, RPT - zrows)
        pltpu.sync_copy(ones_v.at[pl.ds(0, zc)], acc.at[pl.ds(base + zrows, zc)])
        zrows += zc

    def onerow(i, _):
        def onecol(j, _):
            ones_v[i, pl.ds(j * 16, 16)] = jnp.ones((16,), jnp.float32)
            return 0
        return lax.fori_loop(0, D // 16, onecol, 0)
    lax.fori_loop(0, CH, onerow, 0)
    pltpu.sync_copy(packed_hbm.at[wid], idx_v)
    plsc.subcore_barrier()

    def chunk(j, _):
        _decode(idx_v, j, True, None, idst_v)
        pltpu.sync_copy(ones_v, acc.at[idst_v], add=True)
        return 0
    lax.fori_loop(0, NCH, chunk, 0)

    plsc.subcore_barrier()
    pltpu.sync_copy(acc.at[pl.ds(base, RPT)], out_hbm.at[cid, pl.ds(base, RPT)])


@functools.partial(
    pl.kernel,
    mesh=_mesh,
    out_type=jax.ShapeDtypeStruct((2, NPAD, D), jnp.float32),
    scratch_types=[
        pltpu.VMEM((NCH, CH), jnp.int32),
        pltpu.VMEM((CH,), jnp.int32),
        pltpu.VMEM((CH,), jnp.int32),
        pltpu.VMEM((CH,), jnp.int32),
        pltpu.VMEM((CH,), jnp.int32),
        pltpu.VMEM((CH, D), jnp.float32),
        pltpu.VMEM((CH, D), jnp.float32),
        pltpu.VMEM_SHARED((NPAD, D), jnp.float32),
        pltpu.SemaphoreType.DMA,
        pltpu.SemaphoreType.DMA,
    ],
)
def _scat_sc(g_hbm, packed_hbm, out_hbm,
             packed_v, is0, id0, is1, id1, buf0, buf1, acc, sem0, sem1):
    cid = lax.axis_index("c")
    sid = lax.axis_index("s")
    wid = sid * 2 + cid
    _zero_vmem(buf0, CH, D)
    base = sid * RPT
    zrows = 0
    while zrows < RPT:
        zc = min(CH, RPT - zrows)
        pltpu.sync_copy(buf0.at[pl.ds(0, zc)], acc.at[pl.ds(base + zrows, zc)])
        zrows += zc
    pltpu.sync_copy(packed_hbm.at[wid], packed_v)
    plsc.subcore_barrier()

    bufs = (buf0, buf1)
    sems = (sem0, sem1)
    isrcs = (is0, is1)
    idsts = (id0, id1)
    for b in range(2):
        _decode(packed_v, b, False, isrcs[b], idsts[b])
        pltpu.async_copy(g_hbm.at[isrcs[b]], bufs[b], sems[b])

    def step(g, _):
        j0 = g * 2
        for b in range(2):
            j = j0 + b
            pltpu.make_async_copy(g_hbm.at[isrcs[b]], bufs[b], sems[b]).wait()
            pltpu.sync_copy(bufs[b], acc.at[idsts[b]], add=True)
            _decode(packed_v, j + 2, False, isrcs[b], idsts[b])
            pltpu.async_copy(g_hbm.at[isrcs[b]], bufs[b], sems[b])
        return 0
    lax.fori_loop(0, (NCH - 2) // 2, step, 0)
    for b in range(2):
        pltpu.make_async_copy(g_hbm.at[isrcs[b]], bufs[b], sems[b]).wait()
        pltpu.sync_copy(bufs[b], acc.at[idsts[b]], add=True)

    plsc.subcore_barrier()
    pltpu.sync_copy(acc.at[pl.ds(base, RPT)], out_hbm.at[cid, pl.ds(base, RPT)])


BM = 400
GRID = N // BM


def _tc1_body(x_ref, c0_ref, c1_ref, w_ref, b_ref, o_ref, inv_ref):
    cnt = (jnp.max(c0_ref[...], axis=1, keepdims=True)
           + jnp.max(c1_ref[...], axis=1, keepdims=True))
    inv = 1.0 / jnp.sqrt(cnt + 1.0)
    inv_ref[...] = jnp.broadcast_to(inv, inv_ref.shape)
    hw = jnp.dot(x_ref[...], w_ref[...], preferred_element_type=jnp.float32)
    o_ref[...] = (hw + b_ref[...]) * inv


def _tc2_body(p0_ref, p1_ref, g_ref, inv_ref, w_ref, b_ref, o_ref):
    inv = inv_ref[...]
    h1 = jax.nn.relu(inv * (p0_ref[...] + p1_ref[...] + g_ref[...]))
    hw = jnp.dot(h1, w_ref[...], preferred_element_type=jnp.float32)
    o_ref[...] = (hw + b_ref[...]) * inv


def _tc3_body(p0_ref, p1_ref, g_ref, inv_ref,
              wf1_ref, bf1_ref, wf2_ref, bf2_ref, wf3_ref, bf3_ref,
              ws_ref, bs_ref, o_ref):
    hx = inv_ref[...] * (p0_ref[...] + p1_ref[...] + g_ref[...])
    h = jax.nn.relu(jnp.dot(hx, wf1_ref[...], preferred_element_type=jnp.float32) + bf1_ref[...])
    h = jax.nn.relu(jnp.dot(h, wf2_ref[...], preferred_element_type=jnp.float32) + bf2_ref[...])
    h = jax.nn.relu(jnp.dot(h, wf3_ref[...], preferred_element_type=jnp.float32) + bf3_ref[...])
    o_ref[...] = h + jnp.dot(hx, ws_ref[...], preferred_element_type=jnp.float32) + bs_ref[...]


def _row_spec(shape):
    return pl.BlockSpec((BM, shape), lambda i: (i, 0))


def _full_spec(r, c):
    return pl.BlockSpec((r, c), lambda i: (0, 0))


_out_nd = jax.ShapeDtypeStruct((N, D), jnp.float32)
_out_row = pl.BlockSpec((BM, D), lambda i: (i, 0))

_tc1 = pl.pallas_call(
    _tc1_body, grid=(GRID,),
    in_specs=[_row_spec(D), _row_spec(D), _row_spec(D),
              _full_spec(D, D), _full_spec(1, D)],
    out_specs=[_out_row, _out_row], out_shape=[_out_nd, _out_nd])

_tc2 = pl.pallas_call(
    _tc2_body, grid=(GRID,),
    in_specs=[_row_spec(D), _row_spec(D), _row_spec(D), _row_spec(D),
              _full_spec(D, D), _full_spec(1, D)],
    out_specs=_out_row, out_shape=_out_nd)

_tc3 = pl.pallas_call(
    _tc3_body, grid=(GRID,),
    in_specs=[_row_spec(D), _row_spec(D), _row_spec(D), _row_spec(D)]
             + [_full_spec(D, D), _full_spec(1, D)] * 4,
    out_specs=_out_row, out_shape=_out_nd)


def kernel(x, edge_index, W1, b1, W2, b2, Wf1, bf1, Wf2, bf2, Wf3, bf3, Ws, bs):
    pad = EPAD - E
    src = jnp.concatenate([edge_index[0], jnp.zeros((pad,), jnp.int32)])
    dst = jnp.concatenate([edge_index[1], jnp.full((pad,), N, jnp.int32)])
    packed = (src | (dst << 16)).reshape(NW, NCH, CH)

    cnt = _deg_sc(packed)
    c0, c1 = cnt[0], cnt[1]

    b1r = b1.reshape(1, D)
    g1, invb = _tc1(x, c0, c1, W1, b1r)
    P1 = _scat_sc(g1, packed)
    g2 = _tc2(P1[0], P1[1], g1, invb, W2, b2.reshape(1, D))
    P2 = _scat_sc(g2, packed)
    out = _tc3(P2[0], P2[1], g2, invb,
               Wf1, bf1.reshape(1, D), Wf2, bf2.reshape(1, D),
               Wf3, bf3.reshape(1, D), Ws, bs.reshape(1, D))
    return out
